# baseline (speedup 1.0000x reference)
import functools

import jax
import jax.numpy as jnp
from jax.experimental import pallas as pl
from jax.experimental.pallas import tpu as pltpu

POOL = 2
VMEM_LIMIT = 48 * 1024 * 1024
G_CONV1 = 2
G_CONV2 = 4


def _sig(x):
    return pl.reciprocal(1.0 + jnp.exp(-x), approx=True)


def _conv_body(G, hp, kh, N, split_c, x_ref, t_ref, b_ref, o_ref):
    if split_c:
        C = x_ref.shape[0] // G
        planes = [jnp.concatenate(
            [x_ref[g * C + c].astype(jnp.bfloat16) for c in range(C)], axis=1)
            for g in range(G)]
    else:
        planes = [x_ref[g] for g in range(G)]
    mr = POOL * hp
    acc = None
    for i in range(kh):
        rows = [planes[g][i: i + mr] for g in range(G)]
        band = rows[0] if G == 1 else jnp.concatenate(rows, axis=0)
        d = jnp.dot(band, t_ref[i], preferred_element_type=jnp.float32)
        acc = d if acc is None else acc + d
    z = _sig(jnp.maximum(acc[:, :N], acc[:, N:]) + b_ref[...])
    zb = z.astype(jnp.bfloat16)
    mo = G * hp
    mi = G * mr
    row_o = jax.lax.broadcasted_iota(jnp.int32, (2 * mo, mi), 0)
    row_i = jax.lax.broadcasted_iota(jnp.int32, (2 * mo, mi), 1)
    sel = (row_i == 2 * (row_o % mo) + row_o // mo).astype(jnp.bfloat16)
    eo = jnp.dot(sel, zb, preferred_element_type=jnp.float32)
    m = jnp.maximum(eo[:mo], eo[mo:])
    o_ref[...] = m.astype(o_ref.dtype).reshape(G, hp, N)


def _conv_stage(x_in, t_cat, b_row, G, split_c=False, B=None):
    if not split_c:
        B = x_in.shape[0]
    C = x_in.shape[0] // B
    kh, WC, N2 = t_cat.shape
    N = N2 // 2
    H = x_in.shape[1]
    hp = (H - kh + 1) // POOL
    if split_c:
        in_spec = pl.BlockSpec((G * C, H, x_in.shape[2]), lambda i: (i, 0, 0))
    else:
        in_spec = pl.BlockSpec((G, H, WC), lambda i: (i, 0, 0))
    return pl.pallas_call(
        functools.partial(_conv_body, G, hp, kh, N, split_c),
        out_shape=jax.ShapeDtypeStruct((B, hp, N), jnp.bfloat16),
        grid=(B // G,),
        in_specs=[
            in_spec,
            pl.BlockSpec((kh, WC, N2), lambda i: (0, 0, 0)),
            pl.BlockSpec((1, N), lambda i: (0, 0)),
        ],
        out_specs=pl.BlockSpec((G, hp, N), lambda i: (i, 0, 0)),
        compiler_params=pltpu.CompilerParams(
            dimension_semantics=("parallel",),
            vmem_limit_bytes=VMEM_LIMIT),
    )(x_in, t_cat, b_row)


def _fc_body(x_ref, w1_ref, b1_ref, w2_ref, b2_ref, w3_ref, b3_ref, o_ref):
    R, NF = x_ref.shape[1], x_ref.shape[2]
    acc = None
    for r in range(R):
        d = jnp.dot(x_ref[:, r, :], w1_ref[r * NF:(r + 1) * NF, :],
                    preferred_element_type=jnp.float32)
        acc = d if acc is None else acc + d
    h1 = _sig(acc + b1_ref[...])
    h2 = _sig(jnp.dot(h1, w2_ref[...],
                      preferred_element_type=jnp.float32) + b2_ref[...])
    o_ref[...] = (jnp.dot(h2, w3_ref[...],
                          preferred_element_type=jnp.float32) + b3_ref[...])


def _fc_stage(y2, w1, b1, w2, b2, w3, b3):
    MB, R, NF = y2.shape
    H1, H2, NC = w1.shape[1], w2.shape[1], w3.shape[1]
    MT = MB // 2 if MB % 16 == 0 else MB
    return pl.pallas_call(
        _fc_body,
        out_shape=jax.ShapeDtypeStruct((MB, NC), jnp.float32),
        grid=(MB // MT,),
        in_specs=[
            pl.BlockSpec((MT, R, NF), lambda i: (i, 0, 0)),
            pl.BlockSpec((R * NF, H1), lambda i: (0, 0)),
            pl.BlockSpec((1, H1), lambda i: (0, 0)),
            pl.BlockSpec((H1, H2), lambda i: (0, 0)),
            pl.BlockSpec((1, H2), lambda i: (0, 0)),
            pl.BlockSpec((H2, NC), lambda i: (0, 0)),
            pl.BlockSpec((1, NC), lambda i: (0, 0)),
        ],
        out_specs=pl.BlockSpec((MT, NC), lambda i: (i, 0)),
        compiler_params=pltpu.CompilerParams(
            dimension_semantics=("parallel",),
            vmem_limit_bytes=VMEM_LIMIT),
    )(y2, w1, b1.reshape(1, H1), w2, b2.reshape(1, H2), w3, b3.reshape(1, NC))


def _prep2_body(kh, t_ref, o_ref):
    for i in range(kh):
        o_ref[i] = jnp.concatenate(
            [t_ref[0, i].astype(jnp.bfloat16),
             t_ref[1, i].astype(jnp.bfloat16)], axis=1)


def _prep2(t):
    kh, WC, N = t.shape[1], t.shape[2], t.shape[3]
    return pl.pallas_call(
        functools.partial(_prep2_body, kh),
        out_shape=jax.ShapeDtypeStruct((kh, WC, 2 * N), jnp.bfloat16),
        compiler_params=pltpu.CompilerParams(
            vmem_limit_bytes=VMEM_LIMIT),
    )(t)


def _prep1_body(kh, W, C, t_ref, o_ref):
    WC = W * C
    r_out = jax.lax.broadcasted_iota(jnp.int32, (WC, WC), 0)
    r_in = jax.lax.broadcasted_iota(jnp.int32, (WC, WC), 1)
    perm = ((r_out % W) * C + r_out // W == r_in).astype(jnp.bfloat16)
    for i in range(kh):
        pb = [jnp.dot(perm, t_ref[dw, i].astype(jnp.bfloat16),
                      preferred_element_type=jnp.float32).astype(jnp.bfloat16)
              for dw in range(2)]
        o_ref[i] = jnp.concatenate(pb, axis=1)


def _prep1(t, W, C):
    kh, WC, N = t.shape[1], t.shape[2], t.shape[3]
    return pl.pallas_call(
        functools.partial(_prep1_body, kh, W, C),
        out_shape=jax.ShapeDtypeStruct((kh, WC, 2 * N), jnp.bfloat16),
        compiler_params=pltpu.CompilerParams(
            vmem_limit_bytes=VMEM_LIMIT),
    )(t)


def kernel(x, t1, b1, t2, b2, fc1_w, fc1_b, fc2_w, fc2_b, fc3_w, fc3_b):
    B, C, H, W = x.shape

    xp = x.reshape(B * C, H, W)
    y1 = _conv_stage(xp, _prep1(t1, W, C), b1, G_CONV1,
                     split_c=True, B=B)
    y2 = _conv_stage(y1, _prep2(t2), b2, G_CONV2)

    return _fc_stage(y2, fc1_w.astype(jnp.bfloat16), fc1_b,
                     fc2_w, fc2_b, fc3_w, fc3_b)
